# baseline (speedup 1.0000x reference)
import functools

import jax
import jax.numpy as jnp
from jax import lax
from jax.experimental import pallas as pl
from jax.experimental.pallas import tpu as pltpu
from jax.experimental.pallas import tpu_sc as plsc

_H = 384
_W = 384
_MH = 28
_MW = 28
_NW = 32
_LANES = 16
_ZBASE = _MH * 32
_CH = 16
_CW = _CH * _W
_CB = _CW * 4


def _sc_paste(pi, pf, masks, n_tasks):
    mesh = plsc.VectorSubcoreMesh(core_axis_name="c", subcore_axis_name="s")

    @functools.partial(
        pl.kernel,
        out_type=jax.ShapeDtypeStruct((n_tasks * _H * _W,), jnp.float32),
        mesh=mesh,
        compiler_params=pltpu.CompilerParams(needs_layout_passes=False),
        scratch_types=[
            pltpu.VMEM((n_tasks * _LANES,), jnp.int32),
            pltpu.VMEM((n_tasks * _LANES,), jnp.float32),
            pltpu.VMEM((2, _MH * _MW), jnp.float32),
            pltpu.VMEM((_MH * 32 + 32,), jnp.float32),
            pltpu.VMEM((_W,), jnp.int32),
            pltpu.VMEM((_H,), jnp.int32),
            pltpu.VMEM((2 * _CW,), jnp.float32),
            pltpu.VMEM((_CW,), jnp.float32),
            pltpu.SemaphoreType.DMA((2,)),
            pltpu.SemaphoreType.DMA,
            pltpu.SemaphoreType.DMA,
        ],
    )
    def k(pi_hbm, pf_hbm, masks_hbm, out_hbm, pi_v, pf_v, mraw, mpad, cidx_v,
          rbase_v, exp, zbuf, sem_pp, sem_z, sem_m):
        cid = lax.axis_index("c")
        sid = lax.axis_index("s")
        wid = sid * 2 + cid

        pltpu.sync_copy(pi_hbm, pi_v)
        pltpu.sync_copy(pf_hbm, pf_v)

        lane = lax.iota(jnp.int32, _LANES)
        zeros16 = jnp.zeros((_LANES,), jnp.float32)

        def zinit(q, _):
            zbuf[pl.ds(q * _LANES, _LANES)] = zeros16
            return 0

        lax.fori_loop(0, _CW // _LANES, zinit, 0)
        mpad[pl.ds(_ZBASE, _LANES)] = zeros16
        mpad[pl.ds(_ZBASE + _LANES, _LANES)] = zeros16

        n_mine = jnp.where(wid < n_tasks % _NW, 1, 0) + n_tasks // _NW

        pltpu.async_copy(masks_hbm.at[wid], mraw.at[0], sem_m)

        def task(i, carry):
            cnum, nz = carry
            t = wid + i * _NW
            ib = i & 1

            pltpu.make_async_copy(masks_hbm.at[t], mraw.at[ib],
                                  sem_m).wait()

            @pl.when(i + 1 < n_mine)
            def _():
                pltpu.async_copy(masks_hbm.at[t + _NW], mraw.at[1 - ib],
                                 sem_m)

            vi = pi_v[pl.ds(pl.multiple_of(t * _LANES, _LANES), _LANES)]
            vf = pf_v[pl.ds(pl.multiple_of(t * _LANES, _LANES), _LANES)]
            y1 = vi[0]
            x1 = vi[1]
            y2 = vi[2]
            x2 = vi[3]
            ry = vf[4]
            rx = vf[5]

            ibv = jnp.full((_LANES,), ib, jnp.int32)

            def pad_row(r, _):
                v0 = plsc.load_gather(mraw, [ibv, r * _MW + lane])
                mpad[pl.ds(r * 32, _LANES)] = v0
                hi = jnp.minimum(r * _MW + _LANES + lane, _MH * _MW - 1)
                v1 = plsc.load_gather(mraw, [ibv, hi])
                v1 = jnp.where(lane < _MW - _LANES, v1, 0.0)
                mpad[pl.ds(r * 32 + _LANES, _LANES)] = v1
                return 0

            lax.fori_loop(0, _MH, pad_row, 0)

            for j in range(_W // _LANES):
                x = lane + j * _LANES
                cx = x - x1
                ci = (cx.astype(jnp.float32) * rx).astype(jnp.int32)
                ci = jnp.minimum(jnp.maximum(ci, 0), _MW - 1)
                ok = (x >= x1) & (x < x2)
                cidx_v[pl.ds(j * _LANES, _LANES)] = jnp.where(ok, ci, _MW)

            for j in range(_H // _LANES):
                y = lane + j * _LANES
                dy = y - y1
                ri = (dy.astype(jnp.float32) * ry).astype(jnp.int32)
                ri = jnp.minimum(jnp.maximum(ri, 0), _MH - 1)
                ok = (y >= y1) & (y < y2)
                rbase_v[pl.ds(j * _LANES, _LANES)] = jnp.where(
                    ok, ri * 32, _ZBASE)

            obase = t * (_H * _W)
            ya = (y1 // _CH) * _CH
            yb = ((y2 + _CH - 1) // _CH) * _CH

            def ztop(q, _):
                pltpu.async_copy(zbuf,
                                 out_hbm.at[pl.ds(obase + q * _CW, _CW)],
                                 sem_z)
                return 0

            lax.fori_loop(0, ya // _CH, ztop, 0)

            def zbot(q, _):
                pltpu.async_copy(
                    zbuf,
                    out_hbm.at[pl.ds(obase + (yb + q * _CH) * _W, _CW)],
                    sem_z)
                return 0

            lax.fori_loop(0, (_H - yb) // _CH, zbot, 0)

            def chunk(c, cnum):
                slot = cnum & 1

                @pl.when(cnum >= 2)
                def _():
                    pltpu.make_async_copy(
                        exp.at[pl.ds(0, _CW)],
                        out_hbm.at[pl.ds(0, _CW)], sem_pp.at[slot]).wait()

                y0 = ya + c * _CH
                sbase = slot * _CW
                rbv = rbase_v[pl.ds(y0, _LANES)]
                for yy in range(_CH):
                    rb = rbv[yy]
                    for j in range(_W // _LANES):
                        idx = cidx_v[pl.ds(j * _LANES, _LANES)] + rb
                        exp[pl.ds(sbase + yy * _W + j * _LANES,
                                  _LANES)] = plsc.load_gather(mpad, [idx])
                pltpu.async_copy(exp.at[pl.ds(sbase, _CW)],
                                 out_hbm.at[pl.ds(obase + y0 * _W, _CW)],
                                 sem_pp.at[slot])
                return cnum + 1

            cnum = lax.fori_loop(0, (yb - ya) // _CH, chunk, cnum)
            nz = nz + ya // _CH + (_H - yb) // _CH
            return (cnum, nz)

        cnum, nz = lax.fori_loop(0, n_mine, task, (0, 0))

        @pl.when(cnum >= 1)
        def _():
            pltpu.make_async_copy(exp.at[pl.ds(0, _CW)],
                                  out_hbm.at[pl.ds(0, _CW)],
                                  sem_pp.at[(cnum - 1) & 1]).wait()

        @pl.when(cnum >= 2)
        def _():
            pltpu.make_async_copy(exp.at[pl.ds(0, _CW)],
                                  out_hbm.at[pl.ds(0, _CW)],
                                  sem_pp.at[cnum & 1]).wait()

        def drain_z(q, _):
            pltpu.make_async_copy(zbuf, out_hbm.at[pl.ds(0, _CW)],
                                  sem_z).wait()
            return 0

        lax.fori_loop(0, nz, drain_z, 0)

    return k(pi, pf, masks)


def kernel(bboxess, counts, maskss, img_h, img_w):
    B, K = maskss.shape[0], maskss.shape[1]
    n = B * K

    boxes = jnp.round(bboxess).astype(jnp.int32)
    y1 = jnp.clip(boxes[..., 0], 0, img_h - 1)
    x1 = jnp.clip(boxes[..., 1], 0, img_w - 1)
    y2 = jnp.clip(boxes[..., 2], y1 + 1, img_h)
    x2 = jnp.clip(boxes[..., 3], x1 + 1, img_w)
    active = jnp.arange(K, dtype=jnp.int32)[None, :] < counts
    y2 = jnp.where(active, y2, y1)
    ratio_y = _MH / jnp.maximum(y2 - y1, 1).astype(jnp.float32)
    ratio_x = _MW / (x2 - x1).astype(jnp.float32)

    zi = jnp.zeros_like(y1)
    pi = jnp.stack([y1, x1, y2, x2] + [zi] * 12, axis=-1)
    pi = pi.reshape(n * _LANES).astype(jnp.int32)
    zf = jnp.zeros_like(ratio_y)
    pf = jnp.stack([zf, zf, zf, zf, ratio_y, ratio_x] + [zf] * 10, axis=-1)
    pf = pf.reshape(n * _LANES).astype(jnp.float32)
    masks = maskss.reshape(n, _MH * _MW).astype(jnp.float32)

    out = _sc_paste(pi, pf, masks, n)
    return out.reshape(B, K, 1, _H, _W)


# speedup vs baseline: 1.7222x; 1.7222x over previous
import functools

import jax
import jax.numpy as jnp
from jax import lax
from jax.experimental import pallas as pl
from jax.experimental.pallas import tpu as pltpu
from jax.experimental.pallas import tpu_sc as plsc

_H = 384
_W = 384
_MH = 28
_MW = 28
_NW = 32
_LANES = 16
_REP = 4
_RB = _MH * _REP * _W
_ZOFF = 2 * _RB
_ZCHUNK = 16


def _sc_paste(pi, pf, masks, n_tasks):
    mesh = plsc.VectorSubcoreMesh(core_axis_name="c", subcore_axis_name="s")

    @functools.partial(
        pl.kernel,
        out_type=jax.ShapeDtypeStruct((n_tasks * _H * _W,), jnp.float32),
        mesh=mesh,
        compiler_params=pltpu.CompilerParams(needs_layout_passes=False),
        scratch_types=[
            pltpu.VMEM((n_tasks * _LANES,), jnp.int32),
            pltpu.VMEM((n_tasks * _LANES,), jnp.float32),
            pltpu.VMEM((2, _MH * _MW), jnp.float32),
            pltpu.VMEM((_MH * 32,), jnp.float32),
            pltpu.VMEM((_W,), jnp.int32),
            pltpu.VMEM((_H,), jnp.int32),
            pltpu.VMEM((2 * _RB + _ZCHUNK * _W,), jnp.float32),
            pltpu.SemaphoreType.DMA((2,)),
            pltpu.SemaphoreType.DMA,
        ],
    )
    def k(pi_hbm, pf_hbm, masks_hbm, out_hbm, pi_v, pf_v, mraw, mpad, cidx_v,
          ridx_v, rxf, sem, sem_m):
        cid = lax.axis_index("c")
        sid = lax.axis_index("s")
        wid = sid * 2 + cid

        pltpu.sync_copy(pi_hbm, pi_v)
        pltpu.sync_copy(pf_hbm, pf_v)

        lane = lax.iota(jnp.int32, _LANES)
        zeros16 = jnp.zeros((_LANES,), jnp.float32)

        def zinit(q, _):
            rxf[pl.ds(_ZOFF + q * _LANES, _LANES)] = zeros16
            return 0

        lax.fori_loop(0, _ZCHUNK * _W // _LANES, zinit, 0)

        n_mine = jnp.where(wid < n_tasks % _NW, 1, 0) + n_tasks // _NW

        pltpu.async_copy(masks_hbm.at[wid], mraw.at[0], sem_m)

        def task(i, _):
            t = wid + i * _NW
            ib = i & 1
            bb = ib * _RB
            smy = sem.at[ib]

            @pl.when(i >= 2)
            def _():
                def drain(q, _):
                    pltpu.make_async_copy(
                        rxf.at[pl.ds(0, 24 * _W)],
                        out_hbm.at[pl.ds(0, 24 * _W)], smy).wait()
                    return 0

                lax.fori_loop(0, _H // 24, drain, 0)

            pltpu.make_async_copy(masks_hbm.at[t], mraw.at[ib],
                                  sem_m).wait()

            @pl.when(i + 1 < n_mine)
            def _():
                pltpu.async_copy(masks_hbm.at[t + _NW], mraw.at[1 - ib],
                                 sem_m)

            vi = pi_v[pl.ds(pl.multiple_of(t * _LANES, _LANES), _LANES)]
            vf = pf_v[pl.ds(pl.multiple_of(t * _LANES, _LANES), _LANES)]
            y1 = vi[0]
            x1 = vi[1]
            y2 = vi[2]
            x2 = vi[3]
            ry = vf[4]
            rx = vf[5]

            ibv = jnp.full((_LANES,), ib, jnp.int32)

            def pad_row(r, _):
                v0 = plsc.load_gather(mraw, [ibv, r * _MW + lane])
                mpad[pl.ds(r * 32, _LANES)] = v0
                hi = jnp.minimum(r * _MW + _LANES + lane, _MH * _MW - 1)
                v1 = plsc.load_gather(mraw, [ibv, hi])
                v1 = jnp.where(lane < _MW - _LANES, v1, 0.0)
                mpad[pl.ds(r * 32 + _LANES, _LANES)] = v1
                return 0

            lax.fori_loop(0, _MH, pad_row, 0)

            for j in range(_W // _LANES):
                x = lane + j * _LANES
                cx = x - x1
                ci = (cx.astype(jnp.float32) * rx).astype(jnp.int32)
                ci = jnp.minimum(jnp.maximum(ci, 0), _MW - 1)
                ok = (x >= x1) & (x < x2)
                cidx_v[pl.ds(j * _LANES, _LANES)] = jnp.where(ok, ci, _MW)

            for j in range(_H // _LANES):
                y = lane + j * _LANES
                dy = y - y1
                ri = (dy.astype(jnp.float32) * ry).astype(jnp.int32)
                ridx_v[pl.ds(j * _LANES, _LANES)] = jnp.minimum(
                    jnp.maximum(ri, 0), _MH - 1)

            def rx_row(r, _):
                base = r * 32
                rb = bb + r * (_REP * _W)
                for j in range(_W // _LANES):
                    idx = cidx_v[pl.ds(j * _LANES, _LANES)] + base
                    v = plsc.load_gather(mpad, [idx])
                    for p in range(_REP):
                        rxf[pl.ds(rb + p * _W + j * _LANES, _LANES)] = v
                return 0

            lax.fori_loop(0, _MH, rx_row, 0)

            obase = t * (_H * _W)

            nztop = y1 // _ZCHUNK

            def ztop(q, _):
                pltpu.async_copy(
                    rxf.at[pl.ds(_ZOFF, _ZCHUNK * _W)],
                    out_hbm.at[pl.ds(obase + q * (_ZCHUNK * _W),
                                     _ZCHUNK * _W)], smy)
                return 0

            lax.fori_loop(0, nztop, ztop, 0)

            def zfill(y0, rem):
                for nbit in (8, 4, 2, 1):
                    yc = y0

                    @pl.when((rem & nbit) != 0)
                    def _():
                        pltpu.async_copy(
                            rxf.at[pl.ds(_ZOFF, nbit * _W)],
                            out_hbm.at[pl.ds(obase + yc * _W, nbit * _W)],
                            smy)

                    y0 = y0 + jnp.where((rem & nbit) != 0, nbit, 0)

            zfill(nztop * _ZCHUNK, y1 - nztop * _ZCHUNK)

            def runs_cond(y):
                return y < y2

            def runs_body(y):
                r = plsc.load_gather(ridx_v,
                                     [jnp.full((_LANES,), y, jnp.int32)])[0]

                def scan_cond(e):
                    re = plsc.load_gather(
                        ridx_v, [jnp.full((_LANES,), jnp.minimum(e, _H - 1),
                                          jnp.int32)])[0]
                    return (e < y2) & (re == r)

                e = lax.while_loop(scan_cond, lambda e: e + 1, y + 1)
                src = bb + r * (_REP * _W)

                def quads_cond(c):
                    return c[0] + _REP <= e

                def quads_body(c):
                    yq = c[0]
                    pltpu.async_copy(
                        rxf.at[pl.ds(src, _REP * _W)],
                        out_hbm.at[pl.ds(obase + yq * _W, _REP * _W)], smy)
                    return (yq + _REP,)

                (y,) = lax.while_loop(quads_cond, quads_body, (y,))
                rem = e - y

                @pl.when((rem & 2) != 0)
                def _():
                    pltpu.async_copy(rxf.at[pl.ds(src, 2 * _W)],
                                     out_hbm.at[pl.ds(obase + y * _W, 2 * _W)],
                                     smy)

                y = y + jnp.where((rem & 2) != 0, 2, 0)

                @pl.when((rem & 1) != 0)
                def _():
                    pltpu.async_copy(rxf.at[pl.ds(src, _W)],
                                     out_hbm.at[pl.ds(obase + y * _W, _W)],
                                     smy)

                return e

            lax.while_loop(runs_cond, runs_body, y1)

            nzbot = (_H - y2) // _ZCHUNK
            y2r = _H - nzbot * _ZCHUNK
            zfill(y2, y2r - y2)

            def zbot(q, _):
                pltpu.async_copy(
                    rxf.at[pl.ds(_ZOFF, _ZCHUNK * _W)],
                    out_hbm.at[pl.ds(obase + (y2r + q * _ZCHUNK) * _W,
                                     _ZCHUNK * _W)], smy)
                return 0

            lax.fori_loop(0, nzbot, zbot, 0)
            return 0

        lax.fori_loop(0, n_mine, task, 0)

        def drain_tail(q, _):
            pltpu.make_async_copy(rxf.at[pl.ds(0, 24 * _W)],
                                  out_hbm.at[pl.ds(0, 24 * _W)],
                                  sem.at[0]).wait()
            pltpu.make_async_copy(rxf.at[pl.ds(0, 24 * _W)],
                                  out_hbm.at[pl.ds(0, 24 * _W)],
                                  sem.at[1]).wait()
            return 0

        lax.fori_loop(0, _H // 24, drain_tail, 0)

    return k(pi, pf, masks)


def kernel(bboxess, counts, maskss, img_h, img_w):
    B, K = maskss.shape[0], maskss.shape[1]
    n = B * K

    boxes = jnp.round(bboxess).astype(jnp.int32)
    y1 = jnp.clip(boxes[..., 0], 0, img_h - 1)
    x1 = jnp.clip(boxes[..., 1], 0, img_w - 1)
    y2 = jnp.clip(boxes[..., 2], y1 + 1, img_h)
    x2 = jnp.clip(boxes[..., 3], x1 + 1, img_w)
    active = jnp.arange(K, dtype=jnp.int32)[None, :] < counts
    y2 = jnp.where(active, y2, y1)
    ratio_y = _MH / jnp.maximum(y2 - y1, 1).astype(jnp.float32)
    ratio_x = _MW / (x2 - x1).astype(jnp.float32)

    zi = jnp.zeros_like(y1)
    pi = jnp.stack([y1, x1, y2, x2] + [zi] * 12, axis=-1)
    pi = pi.reshape(n * _LANES).astype(jnp.int32)
    zf = jnp.zeros_like(ratio_y)
    pf = jnp.stack([zf, zf, zf, zf, ratio_y, ratio_x] + [zf] * 10, axis=-1)
    pf = pf.reshape(n * _LANES).astype(jnp.float32)
    masks = maskss.reshape(n, _MH * _MW).astype(jnp.float32)

    out = _sc_paste(pi, pf, masks, n)
    return out.reshape(B, K, 1, _H, _W)
